# Initial kernel scaffold; baseline (speedup 1.0000x reference)
import jax
import jax.numpy as jnp
from jax.experimental import pallas as pl


def kernel(original_pos, noise, timesteps, sqrt_alphas_cumprod, sqrt_one_minus_alphas_cumprod):
    raise NotImplementedError("write your pallas kernel here")


# speedup vs baseline: 6.8229x; 6.8229x over previous
import dataclasses
import functools

import jax
import jax.numpy as jnp
from jax import lax
from jax.experimental import pallas as pl
from jax.experimental.pallas import tpu as pltpu
from jax.experimental.pallas import tpu_sc as plsc

_LANES = 16
_BLOCK_ROWS = 2048
_TABLE_PAD = 1024


def kernel(original_pos, noise, timesteps, sqrt_alphas_cumprod,
           sqrt_one_minus_alphas_cumprod):
    n, c = original_pos.shape
    flat = n * c
    ta = jnp.pad(sqrt_alphas_cumprod,
                 (0, _TABLE_PAD - sqrt_alphas_cumprod.shape[0]))
    tb = jnp.pad(sqrt_one_minus_alphas_cumprod,
                 (0, _TABLE_PAD - sqrt_one_minus_alphas_cumprod.shape[0]))
    x = original_pos.reshape(flat)
    nz = noise.reshape(flat)

    mesh = plsc.VectorSubcoreMesh(core_axis_name="c", subcore_axis_name="s")
    cp = pltpu.CompilerParams()
    if "needs_layout_passes" in pltpu.CompilerParams.__dataclass_fields__:
        cp = dataclasses.replace(cp, needs_layout_passes=False)

    @functools.partial(
        pl.kernel,
        out_type=jax.ShapeDtypeStruct((flat,), jnp.float32),
        mesh=mesh,
        compiler_params=cp,
        scratch_types=[
            pltpu.VMEM((_TABLE_PAD,), jnp.float32),
            pltpu.VMEM((_TABLE_PAD,), jnp.float32),
        ],
    )
    def _run(x_hbm, n_hbm, t_hbm, ta_hbm, tb_hbm, o_hbm, ta_v, tb_v):
        pltpu.sync_copy(ta_hbm, ta_v)
        pltpu.sync_copy(tb_hbm, tb_v)

        def body(t_v, x_v, n_v, o_v):
            @pl.loop(0, c * _BLOCK_ROWS, step=_LANES)
            def _(k):
                pos = lax.iota(jnp.int32, _LANES) + k
                row = lax.div(pos, c)
                tv = plsc.load_gather(t_v, [row])
                a = plsc.load_gather(ta_v, [tv])
                b = plsc.load_gather(tb_v, [tv])
                sl = pl.ds(k, _LANES)
                o_v[sl] = a * x_v[sl] + b * n_v[sl]

        pltpu.emit_pipeline(
            body,
            grid=(n // _BLOCK_ROWS,),
            in_specs=[
                pl.BlockSpec((_BLOCK_ROWS,), lambda i: (i,)),
                pl.BlockSpec((c * _BLOCK_ROWS,), lambda i: (i,)),
                pl.BlockSpec((c * _BLOCK_ROWS,), lambda i: (i,)),
            ],
            out_specs=[pl.BlockSpec((c * _BLOCK_ROWS,), lambda i: (i,))],
            core_axis_name=("c", "s"),
            dimension_semantics=(pltpu.PARALLEL,),
        )(t_hbm, x_hbm, n_hbm, o_hbm)

    out = _run(x, nz, timesteps, ta, tb)
    return out.reshape(n, c)


# speedup vs baseline: 165.6604x; 24.2799x over previous
import dataclasses
import functools

import jax
import jax.numpy as jnp
from jax.experimental import pallas as pl
from jax.experimental.pallas import tpu as pltpu
from jax.experimental.pallas import tpu_sc as plsc

_LANES = 16
_BLOCK_ROWS = 2048
_TABLE_PAD = 1024


def kernel(original_pos, noise, timesteps, sqrt_alphas_cumprod,
           sqrt_one_minus_alphas_cumprod):
    n, c = original_pos.shape
    ta = jnp.pad(sqrt_alphas_cumprod,
                 (0, _TABLE_PAD - sqrt_alphas_cumprod.shape[0]))
    tb = jnp.pad(sqrt_one_minus_alphas_cumprod,
                 (0, _TABLE_PAD - sqrt_one_minus_alphas_cumprod.shape[0]))
    xs = [original_pos[:, j] for j in range(c)]
    ns = [noise[:, j] for j in range(c)]

    mesh = plsc.VectorSubcoreMesh(core_axis_name="c", subcore_axis_name="s")
    cp = pltpu.CompilerParams()
    if "needs_layout_passes" in pltpu.CompilerParams.__dataclass_fields__:
        cp = dataclasses.replace(cp, needs_layout_passes=False)

    @functools.partial(
        pl.kernel,
        out_type=[jax.ShapeDtypeStruct((n,), jnp.float32)] * c,
        mesh=mesh,
        compiler_params=cp,
        scratch_types=[
            pltpu.VMEM((_TABLE_PAD,), jnp.float32),
            pltpu.VMEM((_TABLE_PAD,), jnp.float32),
        ],
    )
    def _run(x0, x1, x2, n0, n1, n2, t_hbm, ta_hbm, tb_hbm,
             o0, o1, o2, ta_v, tb_v):
        pltpu.sync_copy(ta_hbm, ta_v)
        pltpu.sync_copy(tb_hbm, tb_v)

        def body(t_v, x0v, x1v, x2v, n0v, n1v, n2v, o0v, o1v, o2v):
            @pl.loop(0, _BLOCK_ROWS, step=_LANES)
            def _(k):
                sl = pl.ds(k, _LANES)
                tv = t_v[sl]
                a = plsc.load_gather(ta_v, [tv])
                b = plsc.load_gather(tb_v, [tv])
                o0v[sl] = a * x0v[sl] + b * n0v[sl]
                o1v[sl] = a * x1v[sl] + b * n1v[sl]
                o2v[sl] = a * x2v[sl] + b * n2v[sl]

        bs = pl.BlockSpec((_BLOCK_ROWS,), lambda i: (i,))
        pltpu.emit_pipeline(
            body,
            grid=(n // _BLOCK_ROWS,),
            in_specs=[bs] * 7,
            out_specs=[bs] * 3,
            core_axis_name=("c", "s"),
            dimension_semantics=(pltpu.PARALLEL,),
        )(t_hbm, x0, x1, x2, n0, n1, n2, o0, o1, o2)

    o = _run(*xs, *ns, timesteps, ta, tb)
    return jnp.stack(o, axis=1)


# speedup vs baseline: 168.1224x; 1.0149x over previous
import dataclasses
import functools

import jax
import jax.numpy as jnp
from jax.experimental import pallas as pl
from jax.experimental.pallas import tpu as pltpu
from jax.experimental.pallas import tpu_sc as plsc

_LANES = 16
_BLOCK_ROWS = 4096
_UNROLL = 4
_TABLE_PAD = 1024


def kernel(original_pos, noise, timesteps, sqrt_alphas_cumprod,
           sqrt_one_minus_alphas_cumprod):
    n, c = original_pos.shape
    ta = jnp.pad(sqrt_alphas_cumprod,
                 (0, _TABLE_PAD - sqrt_alphas_cumprod.shape[0]))
    tb = jnp.pad(sqrt_one_minus_alphas_cumprod,
                 (0, _TABLE_PAD - sqrt_one_minus_alphas_cumprod.shape[0]))
    xs = [original_pos[:, j] for j in range(c)]
    ns = [noise[:, j] for j in range(c)]

    mesh = plsc.VectorSubcoreMesh(core_axis_name="c", subcore_axis_name="s")
    cp = pltpu.CompilerParams()
    if "needs_layout_passes" in pltpu.CompilerParams.__dataclass_fields__:
        cp = dataclasses.replace(cp, needs_layout_passes=False)

    @functools.partial(
        pl.kernel,
        out_type=[jax.ShapeDtypeStruct((n,), jnp.float32)] * c,
        mesh=mesh,
        compiler_params=cp,
        scratch_types=[
            pltpu.VMEM((_TABLE_PAD,), jnp.float32),
            pltpu.VMEM((_TABLE_PAD,), jnp.float32),
        ],
    )
    def _run(x0, x1, x2, n0, n1, n2, t_hbm, ta_hbm, tb_hbm,
             o0, o1, o2, ta_v, tb_v):
        pltpu.sync_copy(ta_hbm, ta_v)
        pltpu.sync_copy(tb_hbm, tb_v)

        def body(t_v, x0v, x1v, x2v, n0v, n1v, n2v, o0v, o1v, o2v):
            @pl.loop(0, _BLOCK_ROWS, step=_LANES * _UNROLL)
            def _(k):
                for u in range(_UNROLL):
                    sl = pl.ds(k + u * _LANES, _LANES)
                    tv = t_v[sl]
                    a = plsc.load_gather(ta_v, [tv])
                    b = plsc.load_gather(tb_v, [tv])
                    o0v[sl] = a * x0v[sl] + b * n0v[sl]
                    o1v[sl] = a * x1v[sl] + b * n1v[sl]
                    o2v[sl] = a * x2v[sl] + b * n2v[sl]

        bs = pl.BlockSpec((_BLOCK_ROWS,), lambda i: (i,))
        pltpu.emit_pipeline(
            body,
            grid=(n // _BLOCK_ROWS,),
            in_specs=[bs] * 7,
            out_specs=[bs] * 3,
            core_axis_name=("c", "s"),
            dimension_semantics=(pltpu.PARALLEL,),
        )(t_hbm, x0, x1, x2, n0, n1, n2, o0, o1, o2)

    o = _run(*xs, *ns, timesteps, ta, tb)
    return jnp.stack(o, axis=1)


# speedup vs baseline: 235.2733x; 1.3994x over previous
import dataclasses
import functools

import jax
import jax.numpy as jnp
from jax.experimental import pallas as pl
from jax.experimental.pallas import tpu as pltpu
from jax.experimental.pallas import tpu_sc as plsc

_LANES = 16
_BLOCK_ROWS = 4096
_UNROLL = 4
_TABLE_PAD = 1024


def kernel(original_pos, noise, timesteps, sqrt_alphas_cumprod,
           sqrt_one_minus_alphas_cumprod):
    n, c = original_pos.shape
    ta = jnp.pad(sqrt_alphas_cumprod,
                 (0, _TABLE_PAD - sqrt_alphas_cumprod.shape[0]))
    tb = jnp.pad(sqrt_one_minus_alphas_cumprod,
                 (0, _TABLE_PAD - sqrt_one_minus_alphas_cumprod.shape[0]))
    xs = [original_pos[:, j] for j in range(c)]
    ns = [noise[:, j] for j in range(c)]

    mesh = plsc.VectorSubcoreMesh(core_axis_name="c", subcore_axis_name="s")
    cp = pltpu.CompilerParams()
    if "needs_layout_passes" in pltpu.CompilerParams.__dataclass_fields__:
        cp = dataclasses.replace(cp, needs_layout_passes=False)

    @functools.partial(
        pl.kernel,
        out_type=[jax.ShapeDtypeStruct((n,), jnp.float32)] * c,
        mesh=mesh,
        compiler_params=cp,
        scratch_types=[
            pltpu.VMEM((_TABLE_PAD,), jnp.float32),
            pltpu.VMEM((_TABLE_PAD,), jnp.float32),
        ],
    )
    def _run(x0, x1, x2, n0, n1, n2, t_hbm, ta_hbm, tb_hbm,
             o0, o1, o2, ta_v, tb_v):
        pltpu.sync_copy(ta_hbm, ta_v)
        pltpu.sync_copy(tb_hbm, tb_v)

        def body(t_v, x0v, x1v, x2v, n0v, n1v, n2v, o0v, o1v, o2v):
            @plsc.parallel_loop(0, _BLOCK_ROWS, step=_LANES, unroll=_UNROLL)
            def _(k):
                sl = pl.ds(k, _LANES)
                tv = t_v[sl]
                a = plsc.load_gather(ta_v, [tv])
                b = plsc.load_gather(tb_v, [tv])
                o0v[sl] = a * x0v[sl] + b * n0v[sl]
                o1v[sl] = a * x1v[sl] + b * n1v[sl]
                o2v[sl] = a * x2v[sl] + b * n2v[sl]

        bs = pl.BlockSpec((_BLOCK_ROWS,), lambda i: (i,))
        pltpu.emit_pipeline(
            body,
            grid=(n // _BLOCK_ROWS,),
            in_specs=[bs] * 7,
            out_specs=[bs] * 3,
            core_axis_name=("c", "s"),
            dimension_semantics=(pltpu.PARALLEL,),
        )(t_hbm, x0, x1, x2, n0, n1, n2, o0, o1, o2)

    o = _run(*xs, *ns, timesteps, ta, tb)
    return jnp.stack(o, axis=1)


# speedup vs baseline: 242.5051x; 1.0307x over previous
import dataclasses
import functools

import jax
import jax.numpy as jnp
from jax.experimental import pallas as pl
from jax.experimental.pallas import tpu as pltpu
from jax.experimental.pallas import tpu_sc as plsc

_LANES = 16
_BLOCK_ROWS = 4096
_UNROLL = 4
_TABLE_PAD = 1024


def kernel(original_pos, noise, timesteps, sqrt_alphas_cumprod,
           sqrt_one_minus_alphas_cumprod):
    n, c = original_pos.shape
    ta = jnp.pad(sqrt_alphas_cumprod,
                 (0, _TABLE_PAD - sqrt_alphas_cumprod.shape[0]))
    tb = jnp.pad(sqrt_one_minus_alphas_cumprod,
                 (0, _TABLE_PAD - sqrt_one_minus_alphas_cumprod.shape[0]))
    xs = [original_pos[:, j] for j in range(c)]
    ns = [noise[:, j] for j in range(c)]

    mesh = plsc.VectorSubcoreMesh(core_axis_name="c", subcore_axis_name="s")
    cp = pltpu.CompilerParams()
    if "needs_layout_passes" in pltpu.CompilerParams.__dataclass_fields__:
        cp = dataclasses.replace(cp, needs_layout_passes=False)

    out_rows = 4 * (n // 128)

    @functools.partial(
        pl.kernel,
        out_type=jax.ShapeDtypeStruct((out_rows, 128), jnp.float32),
        mesh=mesh,
        compiler_params=cp,
        scratch_types=[
            pltpu.VMEM((_TABLE_PAD,), jnp.float32),
            pltpu.VMEM((_TABLE_PAD,), jnp.float32),
        ],
    )
    def _run(x0, x1, x2, n0, n1, n2, t_hbm, ta_hbm, tb_hbm, o2d, ta_v, tb_v):
        pltpu.sync_copy(ta_hbm, ta_v)
        pltpu.sync_copy(tb_hbm, tb_v)

        def body(t_v, x0v, x1v, x2v, n0v, n1v, n2v, o2v):
            @plsc.parallel_loop(0, _BLOCK_ROWS, step=_LANES, unroll=_UNROLL)
            def _(k):
                sl = pl.ds(k, _LANES)
                mm = k // 128
                r = k - mm * 128
                tv = t_v[sl]
                a = plsc.load_gather(ta_v, [tv])
                b = plsc.load_gather(tb_v, [tv])
                o2v[4 * mm + 0, pl.ds(r, _LANES)] = a * x0v[sl] + b * n0v[sl]
                o2v[4 * mm + 1, pl.ds(r, _LANES)] = a * x1v[sl] + b * n1v[sl]
                o2v[4 * mm + 2, pl.ds(r, _LANES)] = a * x2v[sl] + b * n2v[sl]

        bs = pl.BlockSpec((_BLOCK_ROWS,), lambda i: (i,))
        bso = pl.BlockSpec((4 * _BLOCK_ROWS // 128, 128), lambda i: (i, 0))
        pltpu.emit_pipeline(
            body,
            grid=(n // _BLOCK_ROWS,),
            in_specs=[bs] * 7,
            out_specs=[bso],
            core_axis_name=("c", "s"),
            dimension_semantics=(pltpu.PARALLEL,),
        )(t_hbm, x0, x1, x2, n0, n1, n2, o2d)

    o = _run(*xs, *ns, timesteps, ta, tb)
    out = o.reshape(n // 128, 4, 128)[:, :c, :]
    return jnp.swapaxes(out, 1, 2).reshape(n, c)
